# baseline (speedup 1.0000x reference)
import functools

import jax
import jax.numpy as jnp
from jax import lax
from jax.experimental import pallas as pl
from jax.experimental.pallas import tpu as pltpu
from jax.experimental.pallas import tpu_sc as plsc

_LANES = 16
_NC, _NS = 2, 16
_NW = _NC * _NS


def _make_encoder(B, VP, D, LP):
    G = B // _LANES
    mesh = plsc.VectorSubcoreMesh(core_axis_name="c", subcore_axis_name="s")

    @functools.partial(
        pl.kernel,
        mesh=mesh,
        out_type=jax.ShapeDtypeStruct((D, B), jnp.float32),
        compiler_params=pltpu.CompilerParams(
            use_tc_tiling_on_sc=False, needs_layout_passes=False
        ),
        scratch_types=[
            pltpu.VMEM((VP,), jnp.int32),
            pltpu.VMEM((2 * LP, _LANES), jnp.int32),
            pltpu.VMEM((2 * LP, _LANES), jnp.int32),
            pltpu.VMEM((2, B), jnp.float32),
            pltpu.SemaphoreType.DMA,
            pltpu.SemaphoreType.DMA,
        ],
    )
    def enc(ids_hbm, tpk_hbm, out_hbm, tab_v, ids0, ids1, out_v, sem0, sem1):
        t = lax.axis_index("s") * _NC + lax.axis_index("c")
        pltpu.sync_copy(tpk_hbm.at[t], tab_v)

        def descs(blk, buf, sem):
            return pltpu.make_async_copy(
                ids_hbm.at[pl.ds(blk * 2 * LP, 2 * LP)], buf, sem)

        def compute(g, buf, off):
            zf = jnp.zeros((_LANES,), jnp.float32)
            zi = jnp.zeros((_LANES,), jnp.int32)
            K = 4

            def body(j, c):
                new = []
                for k in range(K):
                    a0, a1, nn = c[3 * k:3 * k + 3]
                    idv = buf[off + j + k, pl.ds(0, _LANES)]
                    w = plsc.load_gather(tab_v, [idv])
                    e0, e1 = plsc.unpack(
                        plsc.bitcast(w, jnp.bfloat16),
                        format=plsc.PackFormat.INTERLEAVED,
                        preferred_element_type=jnp.float32,
                    )
                    new += [a0 + e0, a1 + e1, nn + jnp.minimum(idv, 1)]
                return tuple(new)

            res = plsc.parallel_loop(
                0, LP, step=K, unroll=2, carry=(zf, zf, zi) * K)(body)

            a0 = res[0] + res[3] + res[6] + res[9]
            a1 = res[1] + res[4] + res[7] + res[10]
            nn = res[2] + res[5] + res[8] + res[11]

            scale = 1.0 / jnp.maximum(nn.astype(jnp.float32), 1.0)
            out_v[0, pl.ds(g * _LANES, _LANES)] = a0 * scale
            out_v[1, pl.ds(g * _LANES, _LANES)] = a1 * scale

        NBLK = G // 2
        descs(0, ids0, sem0).start()
        descs(1, ids1, sem1).start()

        @pl.loop(0, NBLK - 2, step=2)
        def _blk(b):
            descs(b, ids0, sem0).wait()
            compute(2 * b, ids0, 0)
            compute(2 * b + 1, ids0, LP)
            descs(b + 2, ids0, sem0).start()
            descs(b + 1, ids1, sem1).wait()
            compute(2 * b + 2, ids1, 0)
            compute(2 * b + 3, ids1, LP)
            descs(b + 3, ids1, sem1).start()

        descs(NBLK - 2, ids0, sem0).wait()
        compute(2 * NBLK - 4, ids0, 0)
        compute(2 * NBLK - 3, ids0, LP)
        descs(NBLK - 1, ids1, sem1).wait()
        compute(2 * NBLK - 2, ids1, 0)
        compute(2 * NBLK - 1, ids1, LP)

        pltpu.sync_copy(out_v, out_hbm.at[pl.ds(t * 2, 2)])

    return enc


def kernel(ids, table):
    B, S = ids.shape
    V, D = table.shape
    LP = ((S + _LANES - 1) // _LANES) * _LANES
    VP = ((V + 7) // 8) * 8
    ids_p = ids.astype(jnp.int32)
    if LP != S:
        ids_p = jnp.pad(ids_p, ((0, 0), (0, LP - S)))
    ids_t = ids_p.reshape(B // _LANES, _LANES, LP).swapaxes(1, 2)
    ids_t = ids_t.reshape(B // _LANES * LP, _LANES)
    tb = table.astype(jnp.bfloat16)
    tb = tb.at[0].set(jnp.bfloat16(0))
    if VP != V:
        tb = jnp.pad(tb, ((0, VP - V), (0, 0)))
    tpk = jax.lax.bitcast_convert_type(tb.reshape(VP, D // 2, 2), jnp.int32)
    tpk_t = tpk.swapaxes(0, 1)

    enc = _make_encoder(B, VP, D, LP)
    out_t = enc(ids_t, tpk_t)
    return out_t.T
